# baseline (speedup 1.0000x reference)
import dataclasses
import functools
import jax
import jax.numpy as jnp
from jax import lax
from jax.experimental import pallas as pl
from jax.experimental.pallas import tpu as pltpu
from jax.experimental.pallas import tpu_sc as plsc

N = 10000
E = 160000
D = 256
LRELU = 0.2

NC, NS, L = 2, 16, 16
NW = NC * NS
CHUNK = 1408
NCH = 114
E_PAD = NCH * CHUNK
N_PAD = 10240
CPS = 8

_mesh = plsc.VectorSubcoreMesh(core_axis_name="c", subcore_axis_name="s")

_sc_params = pltpu.CompilerParams()
if "needs_layout_passes" in pltpu.CompilerParams.__dataclass_fields__:
    _sc_params = dataclasses.replace(_sc_params, needs_layout_passes=False)
_sc_params_untiled = dataclasses.replace(_sc_params, use_tc_tiling_on_sc=False)


def _wid():
    return lax.axis_index("s") * NC + lax.axis_index("c")


def _tc_proj_body(x_ref, w_ref, asrc_ref, adst_ref, es_ref, ed_ref):
    wa = jnp.dot(w_ref[...], asrc_ref[...],
                 preferred_element_type=jnp.float32,
                 precision=lax.Precision.HIGHEST)
    wb = jnp.dot(w_ref[...], adst_ref[...],
                 preferred_element_type=jnp.float32,
                 precision=lax.Precision.HIGHEST)
    x = x_ref[...]
    es_ref[...] = jnp.dot(x, wa, preferred_element_type=jnp.float32,
                          precision=lax.Precision.HIGHEST)
    ed_ref[...] = jnp.dot(x, wb, preferred_element_type=jnp.float32,
                          precision=lax.Precision.HIGHEST)


def _tc_proj(x, w, asrc, adst):
    blk = 1024
    return pl.pallas_call(
        _tc_proj_body,
        grid=(N_PAD // blk,),
        in_specs=[
            pl.BlockSpec((blk, D), lambda i: (i, 0)),
            pl.BlockSpec((D, D), lambda i: (0, 0)),
            pl.BlockSpec((D, 1), lambda i: (0, 0)),
            pl.BlockSpec((D, 1), lambda i: (0, 0)),
        ],
        out_specs=[
            pl.BlockSpec((blk, 1), lambda i: (i, 0)),
            pl.BlockSpec((blk, 1), lambda i: (i, 0)),
        ],
        out_shape=[
            jax.ShapeDtypeStruct((N, 1), jnp.float32),
            jax.ShapeDtypeStruct((N, 1), jnp.float32),
        ],
    )(x, w, asrc, adst)


def _tc_reduce_body(dpart_ref, den_ref):
    den_ref[...] = jnp.sum(dpart_ref[...], axis=0, keepdims=True)


def _tc_reduce(dpart):
    blk = 1024
    return pl.pallas_call(
        _tc_reduce_body,
        grid=(N_PAD // blk,),
        in_specs=[pl.BlockSpec((NW, blk), lambda i: (0, i))],
        out_specs=pl.BlockSpec((1, blk), lambda i: (0, i)),
        out_shape=jax.ShapeDtypeStruct((1, N_PAD), jnp.float32),
    )(dpart)


def _tc_head_body(x_ref, w_ref, h_ref, ht_ref):
    h = jnp.dot(x_ref[...], w_ref[...],
                preferred_element_type=jnp.float32,
                precision=lax.Precision.HIGHEST)
    h_ref[...] = h
    ht_ref[...] = h.T


def _tc_head(x, w):
    blk = 1024
    return pl.pallas_call(
        _tc_head_body,
        grid=(N_PAD // blk,),
        in_specs=[
            pl.BlockSpec((blk, D), lambda i: (i, 0)),
            pl.BlockSpec((D, D), lambda i: (0, 0)),
        ],
        out_specs=[
            pl.BlockSpec((blk, D), lambda i: (i, 0)),
            pl.BlockSpec((D, blk), lambda i: (0, i)),
        ],
        out_shape=[
            jax.ShapeDtypeStruct((N, D), jnp.float32),
            jax.ShapeDtypeStruct((D, N_PAD), jnp.float32),
        ],
    )(x, w)


_NJ = -(-NCH // NW)


def _sc_edge_body(src_hbm, dst_hbm, mw_hbm, es_hbm, ed_hbm,
                  ex_hbm, pk_hbm, dpart_hbm, cpart_hbm,
                  es_v, ed_v, den_v, cnt_v, *rest):
    bufs = [rest[5 * j:5 * j + 5] for j in range(_NJ)]
    sem_t = rest[5 * _NJ]
    sems_i = rest[5 * _NJ + 1:5 * _NJ + 1 + _NJ]
    sem_o = rest[5 * _NJ + 1 + _NJ]
    wid = _wid()

    pltpu.async_copy(es_hbm, es_v, sem_t)
    pltpu.async_copy(ed_hbm, ed_v, sem_t)

    for j in range(_NJ):
        cid = wid + j * NW
        src_v, dst_v, mw_v, _, _ = bufs[j]

        @pl.when(cid < NCH)
        def _():
            base = cid * CHUNK
            pltpu.async_copy(src_hbm.at[pl.ds(base, CHUNK)], src_v, sems_i[j])
            pltpu.async_copy(dst_hbm.at[pl.ds(base, CHUNK)], dst_v, sems_i[j])
            pltpu.async_copy(mw_hbm.at[pl.ds(base, CHUNK)], mw_v, sems_i[j])

    @pl.loop(0, N_PAD, step=L)
    def _(i):
        zero = jnp.zeros((L,), jnp.float32)
        den_v[pl.ds(i, L)] = zero
        cnt_v[pl.ds(i, L)] = zero

    pltpu.make_async_copy(es_hbm, es_v, sem_t).wait()
    pltpu.make_async_copy(ed_hbm, ed_v, sem_t).wait()

    for j in range(_NJ):
        cid = wid + j * NW
        src_v, dst_v, mw_v, ex_v, pk_v = bufs[j]

        @pl.when(cid < NCH)
        def _():
            base = cid * CHUNK
            pltpu.make_async_copy(src_hbm.at[pl.ds(base, CHUNK)], src_v,
                                  sems_i[j]).wait()
            pltpu.make_async_copy(dst_hbm.at[pl.ds(base, CHUNK)], dst_v,
                                  sems_i[j]).wait()
            pltpu.make_async_copy(mw_hbm.at[pl.ds(base, CHUNK)], mw_v,
                                  sems_i[j]).wait()

            @plsc.parallel_loop(0, CHUNK, step=L)
            def _(i):
                sv = src_v[pl.ds(i, L)]
                dv = dst_v[pl.ds(i, L)]
                pk_v[pl.ds(i, L)] = sv * 16384 + dv
                e = plsc.load_gather(es_v, [sv]) + plsc.load_gather(ed_v, [dv])
                e = jnp.where(e > 0, e, LRELU * e)
                ex = jnp.exp(mw_v[pl.ds(i, L)] * e)
                ex_v[pl.ds(i, L)] = ex
                plsc.addupdate_scatter(den_v, [sv], ex)
                plsc.addupdate_scatter(cnt_v, [sv], jnp.ones((L,), jnp.float32))

            pltpu.async_copy(ex_v, ex_hbm.at[pl.ds(base, CHUNK)], sem_o)
            pltpu.async_copy(pk_v, pk_hbm.at[pl.ds(base, CHUNK)], sem_o)

    pltpu.sync_copy(den_v, dpart_hbm.at[wid])
    pltpu.sync_copy(cnt_v, cpart_hbm.at[wid])

    for j in range(_NJ):
        cid = wid + j * NW
        _, _, _, ex_v, pk_v = bufs[j]

        @pl.when(cid < NCH)
        def _():
            base = cid * CHUNK
            pltpu.make_async_copy(ex_v, ex_hbm.at[pl.ds(base, CHUNK)],
                                  sem_o).wait()
            pltpu.make_async_copy(pk_v, pk_hbm.at[pl.ds(base, CHUNK)],
                                  sem_o).wait()


def _sc_edge(src, dst, mw, es, ed):
    buf_types = []
    for _ in range(_NJ):
        buf_types += [
            pltpu.VMEM((CHUNK,), jnp.int32),
            pltpu.VMEM((CHUNK,), jnp.int32),
            pltpu.VMEM((CHUNK,), jnp.float32),
            pltpu.VMEM((CHUNK,), jnp.float32),
            pltpu.VMEM((CHUNK,), jnp.int32),
        ]
    return pl.kernel(
        _sc_edge_body,
        out_type=[
            jax.ShapeDtypeStruct((E_PAD,), jnp.float32),
            jax.ShapeDtypeStruct((E_PAD,), jnp.int32),
            jax.ShapeDtypeStruct((NW, N_PAD), jnp.float32),
            jax.ShapeDtypeStruct((NW, N_PAD), jnp.float32),
        ],
        mesh=_mesh,
        compiler_params=_sc_params,
        scratch_types=[
            pltpu.VMEM((N_PAD,), jnp.float32),
            pltpu.VMEM((N_PAD,), jnp.float32),
            pltpu.VMEM((N_PAD,), jnp.float32),
            pltpu.VMEM((N_PAD,), jnp.float32),
        ] + buf_types + [pltpu.SemaphoreType.DMA] * (_NJ + 2),
    )(src, dst, mw, es, ed)


def _sc_alpha_body(ex_hbm, pk_hbm, den_hbm, ht_hbm, al_hbm, hpk_hbm,
                   den_v, pk_v, ex_v, al_v, a_v, b_v, q_v):
    wid = _wid()
    pltpu.sync_copy(den_hbm.at[0], den_v)

    for j in range(pl.cdiv(NCH, NW)):
        cid = wid + j * NW

        @pl.when(cid < NCH)
        def _():
            base = cid * CHUNK
            pltpu.sync_copy(pk_hbm.at[pl.ds(base, CHUNK)], pk_v)
            pltpu.sync_copy(ex_hbm.at[pl.ds(base, CHUNK)], ex_v)

            @plsc.parallel_loop(0, CHUNK, step=L)
            def _(i):
                sv = lax.shift_right_logical(pk_v[pl.ds(i, L)], 14)
                dv = plsc.load_gather(den_v, [sv])
                al_v[pl.ds(i, L)] = ex_v[pl.ds(i, L)] / (dv + 1e-16)

            pltpu.sync_copy(al_v, al_hbm.at[pl.ds(base, CHUNK)])

    for r in range(CPS // 2):
        k = wid * (CPS // 2) + r
        pltpu.sync_copy(ht_hbm.at[2 * k], a_v)
        pltpu.sync_copy(ht_hbm.at[2 * k + 1], b_v)

        @plsc.parallel_loop(0, N_PAD, step=L)
        def _(i):
            pkd = plsc.pack(a_v[pl.ds(i, L)], b_v[pl.ds(i, L)],
                            format=plsc.PackFormat.INTERLEAVED)
            q_v[pl.ds(i, L)] = plsc.bitcast(pkd, jnp.int32)

        pltpu.sync_copy(q_v, hpk_hbm.at[k])


def _sc_alpha(ex, pk, den, ht):
    return pl.kernel(
        _sc_alpha_body,
        out_type=[
            jax.ShapeDtypeStruct((E_PAD,), jnp.float32),
            jax.ShapeDtypeStruct((D // 2, N_PAD), jnp.int32),
        ],
        mesh=_mesh,
        compiler_params=_sc_params,
        scratch_types=[
            pltpu.VMEM((N_PAD,), jnp.float32),
            pltpu.VMEM((CHUNK,), jnp.int32),
            pltpu.VMEM((CHUNK,), jnp.float32),
            pltpu.VMEM((CHUNK,), jnp.float32),
            pltpu.VMEM((N_PAD,), jnp.float32),
            pltpu.VMEM((N_PAD,), jnp.float32),
            pltpu.VMEM((N_PAD,), jnp.int32),
        ],
    )(ex, pk, den, ht)


def _sc_agg_body(pk_hbm, al_hbm, hpk_hbm, zt_hbm,
                 h_v, z_v, pa_v, aa_v, pb_v, ab_v,
                 sem_a, sem_b):

    wid = _wid()

    def issue(cid, pv, av, sem):
        base = cid * CHUNK
        pltpu.async_copy(pk_hbm.at[pl.ds(base, CHUNK)], pv, sem)
        pltpu.async_copy(al_hbm.at[pl.ds(base, CHUNK)], av, sem)

    def drain(cid, pv, av, sem):
        base = cid * CHUNK
        pltpu.make_async_copy(pk_hbm.at[pl.ds(base, CHUNK)], pv, sem).wait()
        pltpu.make_async_copy(al_hbm.at[pl.ds(base, CHUNK)], av, sem).wait()

    def process(pv_ref, av_ref):
        @plsc.parallel_loop(0, CHUNK, step=L, unroll=4)
        def _(i):
            pv = pv_ref[pl.ds(i, L)]
            sv = lax.shift_right_logical(pv, 14)
            dv = pv & 16383
            av = av_ref[pl.ds(i, L)]
            for c2 in range(CPS // 2):
                cvec = jnp.full((L,), c2, jnp.int32)
                g32 = plsc.load_gather(h_v, [cvec, dv])
                ha, hb = plsc.unpack(plsc.bitcast(g32, jnp.bfloat16),
                                     format=plsc.PackFormat.INTERLEAVED)
                plsc.addupdate_scatter(z_v, [sv + (2 * c2) * N_PAD], ha * av)
                plsc.addupdate_scatter(z_v, [sv + (2 * c2 + 1) * N_PAD],
                                       hb * av)

    c0 = wid * CPS
    pltpu.sync_copy(hpk_hbm.at[pl.ds(wid * (CPS // 2), CPS // 2)], h_v)

    @pl.loop(0, N_PAD * CPS, step=L)
    def _(i):
        z_v[pl.ds(i, L)] = jnp.zeros((L,), jnp.float32)

    issue(0, pa_v, aa_v, sem_a)

    @pl.loop(0, NCH, step=2)
    def _(ci):
        issue(ci + 1, pb_v, ab_v, sem_b)
        drain(ci, pa_v, aa_v, sem_a)
        process(pa_v, aa_v)

        @pl.when(ci + 2 < NCH)
        def _():
            issue(ci + 2, pa_v, aa_v, sem_a)

        drain(ci + 1, pb_v, ab_v, sem_b)
        process(pb_v, ab_v)

    for c in range(CPS):
        pltpu.sync_copy(z_v.at[pl.ds(c * N_PAD, N_PAD)], zt_hbm.at[c0 + c])


def _sc_agg(pk, alpha, hpk):
    return pl.kernel(
        _sc_agg_body,
        out_type=jax.ShapeDtypeStruct((D, N_PAD), jnp.float32),
        mesh=_mesh,
        compiler_params=_sc_params_untiled,
        scratch_types=[
            pltpu.VMEM((CPS // 2, N_PAD), jnp.int32),
            pltpu.VMEM((N_PAD * CPS,), jnp.float32),
            pltpu.VMEM((CHUNK,), jnp.int32),
            pltpu.VMEM((CHUNK,), jnp.float32),
            pltpu.VMEM((CHUNK,), jnp.int32),
            pltpu.VMEM((CHUNK,), jnp.float32),
            pltpu.SemaphoreType.DMA,
            pltpu.SemaphoreType.DMA,
        ],
    )(pk, alpha, hpk)


def _tc_tail_body(zt_ref, x_ref, h_ref, cnt_ref, g_ref, b_ref, o_ref):
    z = zt_ref[...].T
    cnt = jnp.sum(cnt_ref[...].T, axis=1, keepdims=True)
    zb = jnp.where(cnt > 0.0, z, h_ref[...])
    zr = zb + x_ref[...]
    mu = jnp.mean(zr, axis=1, keepdims=True)
    d = zr - mu
    var = jnp.mean(d * d, axis=1, keepdims=True)
    zn = d * lax.rsqrt(var + 1e-5) * g_ref[...] + b_ref[...]
    o_ref[...] = jnp.where(zn > 0.0, zn, jnp.exp(zn) - 1.0)


def _tc_tail(zt, x, h, cpart, gamma, beta):
    blk = 1024
    return pl.pallas_call(
        _tc_tail_body,
        grid=(N_PAD // blk,),
        in_specs=[
            pl.BlockSpec((D, blk), lambda i: (0, i)),
            pl.BlockSpec((blk, D), lambda i: (i, 0)),
            pl.BlockSpec((blk, D), lambda i: (i, 0)),
            pl.BlockSpec((NW, blk), lambda i: (0, i)),
            pl.BlockSpec((1, D), lambda i: (0, 0)),
            pl.BlockSpec((1, D), lambda i: (0, 0)),
        ],
        out_specs=pl.BlockSpec((blk, D), lambda i: (i, 0)),
        out_shape=jax.ShapeDtypeStruct((N, D), jnp.float32),
    )(zt, x, h, cpart, gamma, beta)


@jax.jit
def kernel(X, M_ei, M_w, W, a_src, a_dst, gamma, beta):
    src = M_ei[0]
    dst = M_ei[1]
    pad = E_PAD - E
    src_p = jnp.concatenate([src, jnp.full((pad,), N, jnp.int32)])
    dst_p = jnp.concatenate([dst, jnp.zeros((pad,), jnp.int32)])
    mw_p = jnp.concatenate([M_w, jnp.zeros((pad,), jnp.float32)])

    es, ed = _tc_proj(X, W, a_src[:, None], a_dst[:, None])
    H, HT = _tc_head(X, W)
    es_p = jnp.pad(es[:, 0], (0, N_PAD - N))
    ed_p = jnp.pad(ed[:, 0], (0, N_PAD - N))

    ex, pk, dpart, cpart = _sc_edge(src_p, dst_p, mw_p, es_p, ed_p)
    den = _tc_reduce(dpart)
    alpha, HPK = _sc_alpha(ex, pk, den, HT)
    zt = _sc_agg(pk, alpha, HPK)
    return _tc_tail(zt, X, H, cpart, gamma[None, :], beta[None, :])


# speedup vs baseline: 1.1312x; 1.1312x over previous
import dataclasses
import functools
import jax
import jax.numpy as jnp
from jax import lax
from jax.experimental import pallas as pl
from jax.experimental.pallas import tpu as pltpu
from jax.experimental.pallas import tpu_sc as plsc

N = 10000
E = 160000
D = 256
LRELU = 0.2

NC, NS, L = 2, 16, 16
NW = NC * NS
CHUNK = 1408
NCH = 114
E_PAD = NCH * CHUNK
N_PAD = 10240
CPS = 8

_mesh = plsc.VectorSubcoreMesh(core_axis_name="c", subcore_axis_name="s")

_sc_params = pltpu.CompilerParams()
if "needs_layout_passes" in pltpu.CompilerParams.__dataclass_fields__:
    _sc_params = dataclasses.replace(_sc_params, needs_layout_passes=False)
_sc_params_untiled = dataclasses.replace(_sc_params, use_tc_tiling_on_sc=False)


def _wid():
    return lax.axis_index("s") * NC + lax.axis_index("c")


def _tc_proj_body(x_ref, w_ref, asrc_ref, adst_ref, es_ref, ed_ref):
    wa = jnp.dot(w_ref[...], asrc_ref[...],
                 preferred_element_type=jnp.float32,
                 precision=lax.Precision.HIGHEST)
    wb = jnp.dot(w_ref[...], adst_ref[...],
                 preferred_element_type=jnp.float32,
                 precision=lax.Precision.HIGHEST)
    x = x_ref[...]
    es_ref[...] = jnp.dot(x, wa, preferred_element_type=jnp.float32,
                          precision=lax.Precision.HIGHEST)
    ed_ref[...] = jnp.dot(x, wb, preferred_element_type=jnp.float32,
                          precision=lax.Precision.HIGHEST)


def _tc_proj(x, w, asrc, adst):
    blk = 1024
    return pl.pallas_call(
        _tc_proj_body,
        grid=(N_PAD // blk,),
        in_specs=[
            pl.BlockSpec((blk, D), lambda i: (i, 0)),
            pl.BlockSpec((D, D), lambda i: (0, 0)),
            pl.BlockSpec((D, 1), lambda i: (0, 0)),
            pl.BlockSpec((D, 1), lambda i: (0, 0)),
        ],
        out_specs=[
            pl.BlockSpec((blk, 1), lambda i: (i, 0)),
            pl.BlockSpec((blk, 1), lambda i: (i, 0)),
        ],
        out_shape=[
            jax.ShapeDtypeStruct((N, 1), jnp.float32),
            jax.ShapeDtypeStruct((N, 1), jnp.float32),
        ],
    )(x, w, asrc, adst)


def _tc_reduce_body(dpart_ref, den_ref):
    den_ref[...] = jnp.sum(dpart_ref[...], axis=0, keepdims=True)


def _tc_reduce(dpart):
    blk = 1024
    return pl.pallas_call(
        _tc_reduce_body,
        grid=(N_PAD // blk,),
        in_specs=[pl.BlockSpec((NW, blk), lambda i: (0, i))],
        out_specs=pl.BlockSpec((1, blk), lambda i: (0, i)),
        out_shape=jax.ShapeDtypeStruct((1, N_PAD), jnp.float32),
    )(dpart)


def _tc_head_body(x_ref, w_ref, h_ref, ht_ref):
    h = jnp.dot(x_ref[...], w_ref[...],
                preferred_element_type=jnp.float32,
                precision=lax.Precision.HIGHEST)
    h_ref[...] = h
    ht_ref[...] = h.T


def _tc_head(x, w):
    blk = 1024
    return pl.pallas_call(
        _tc_head_body,
        grid=(N_PAD // blk,),
        in_specs=[
            pl.BlockSpec((blk, D), lambda i: (i, 0)),
            pl.BlockSpec((D, D), lambda i: (0, 0)),
        ],
        out_specs=[
            pl.BlockSpec((blk, D), lambda i: (i, 0)),
            pl.BlockSpec((D, blk), lambda i: (0, i)),
        ],
        out_shape=[
            jax.ShapeDtypeStruct((N, D), jnp.float32),
            jax.ShapeDtypeStruct((D, N_PAD), jnp.float32),
        ],
    )(x, w)


_NJ = -(-NCH // NW)


def _sc_edge_body(src_hbm, dst_hbm, mw_hbm, es_hbm, ed_hbm,
                  ex_hbm, pk_hbm, dpart_hbm, cpart_hbm,
                  es_v, ed_v, den_v, cnt_v, *rest):
    bufs = [rest[5 * j:5 * j + 5] for j in range(_NJ)]
    sem_t = rest[5 * _NJ]
    sems_i = rest[5 * _NJ + 1:5 * _NJ + 1 + _NJ]
    sem_o = rest[5 * _NJ + 1 + _NJ]
    wid = _wid()

    pltpu.async_copy(es_hbm, es_v, sem_t)
    pltpu.async_copy(ed_hbm, ed_v, sem_t)

    for j in range(_NJ):
        cid = wid + j * NW
        src_v, dst_v, mw_v, _, _ = bufs[j]

        @pl.when(cid < NCH)
        def _():
            base = cid * CHUNK
            pltpu.async_copy(src_hbm.at[pl.ds(base, CHUNK)], src_v, sems_i[j])
            pltpu.async_copy(dst_hbm.at[pl.ds(base, CHUNK)], dst_v, sems_i[j])
            pltpu.async_copy(mw_hbm.at[pl.ds(base, CHUNK)], mw_v, sems_i[j])

    @pl.loop(0, N_PAD, step=L)
    def _(i):
        zero = jnp.zeros((L,), jnp.float32)
        den_v[pl.ds(i, L)] = zero
        cnt_v[pl.ds(i, L)] = zero

    pltpu.make_async_copy(es_hbm, es_v, sem_t).wait()
    pltpu.make_async_copy(ed_hbm, ed_v, sem_t).wait()

    for j in range(_NJ):
        cid = wid + j * NW
        src_v, dst_v, mw_v, ex_v, pk_v = bufs[j]

        @pl.when(cid < NCH)
        def _():
            base = cid * CHUNK
            pltpu.make_async_copy(src_hbm.at[pl.ds(base, CHUNK)], src_v,
                                  sems_i[j]).wait()
            pltpu.make_async_copy(dst_hbm.at[pl.ds(base, CHUNK)], dst_v,
                                  sems_i[j]).wait()
            pltpu.make_async_copy(mw_hbm.at[pl.ds(base, CHUNK)], mw_v,
                                  sems_i[j]).wait()

            @plsc.parallel_loop(0, CHUNK, step=L)
            def _(i):
                sv = src_v[pl.ds(i, L)]
                dv = dst_v[pl.ds(i, L)]
                pk_v[pl.ds(i, L)] = sv * 16384 + dv
                e = plsc.load_gather(es_v, [sv]) + plsc.load_gather(ed_v, [dv])
                e = jnp.where(e > 0, e, LRELU * e)
                ex = jnp.exp(mw_v[pl.ds(i, L)] * e)
                ex_v[pl.ds(i, L)] = ex
                plsc.addupdate_scatter(den_v, [sv], ex)
                plsc.addupdate_scatter(cnt_v, [sv], jnp.ones((L,), jnp.float32))

            pltpu.async_copy(ex_v, ex_hbm.at[pl.ds(base, CHUNK)], sem_o)
            pltpu.async_copy(pk_v, pk_hbm.at[pl.ds(base, CHUNK)], sem_o)

    pltpu.sync_copy(den_v, dpart_hbm.at[wid])
    pltpu.sync_copy(cnt_v, cpart_hbm.at[wid])

    for j in range(_NJ):
        cid = wid + j * NW
        _, _, _, ex_v, pk_v = bufs[j]

        @pl.when(cid < NCH)
        def _():
            base = cid * CHUNK
            pltpu.make_async_copy(ex_v, ex_hbm.at[pl.ds(base, CHUNK)],
                                  sem_o).wait()
            pltpu.make_async_copy(pk_v, pk_hbm.at[pl.ds(base, CHUNK)],
                                  sem_o).wait()


def _sc_edge(src, dst, mw, es, ed):
    buf_types = []
    for _ in range(_NJ):
        buf_types += [
            pltpu.VMEM((CHUNK,), jnp.int32),
            pltpu.VMEM((CHUNK,), jnp.int32),
            pltpu.VMEM((CHUNK,), jnp.float32),
            pltpu.VMEM((CHUNK,), jnp.float32),
            pltpu.VMEM((CHUNK,), jnp.int32),
        ]
    return pl.kernel(
        _sc_edge_body,
        out_type=[
            jax.ShapeDtypeStruct((E_PAD,), jnp.float32),
            jax.ShapeDtypeStruct((E_PAD,), jnp.int32),
            jax.ShapeDtypeStruct((NW, N_PAD), jnp.float32),
            jax.ShapeDtypeStruct((NW, N_PAD), jnp.float32),
        ],
        mesh=_mesh,
        compiler_params=_sc_params,
        scratch_types=[
            pltpu.VMEM((N_PAD,), jnp.float32),
            pltpu.VMEM((N_PAD,), jnp.float32),
            pltpu.VMEM((N_PAD,), jnp.float32),
            pltpu.VMEM((N_PAD,), jnp.float32),
        ] + buf_types + [pltpu.SemaphoreType.DMA] * (_NJ + 2),
    )(src, dst, mw, es, ed)


def _sc_pack_body(ht_hbm, hpk_hbm, a_v, b_v, q_v):
    wid = _wid()
    for r in range(CPS // 2):
        k = wid * (CPS // 2) + r
        pltpu.sync_copy(ht_hbm.at[2 * k], a_v)
        pltpu.sync_copy(ht_hbm.at[2 * k + 1], b_v)

        @plsc.parallel_loop(0, N_PAD, step=L)
        def _(i):
            pkd = plsc.pack(a_v[pl.ds(i, L)], b_v[pl.ds(i, L)],
                            format=plsc.PackFormat.INTERLEAVED)
            q_v[pl.ds(i, L)] = plsc.bitcast(pkd, jnp.int32)

        pltpu.sync_copy(q_v, hpk_hbm.at[k])


def _sc_pack(ht):
    return pl.kernel(
        _sc_pack_body,
        out_type=jax.ShapeDtypeStruct((D // 2, N_PAD), jnp.int32),
        mesh=_mesh,
        compiler_params=_sc_params,
        scratch_types=[
            pltpu.VMEM((N_PAD,), jnp.float32),
            pltpu.VMEM((N_PAD,), jnp.float32),
            pltpu.VMEM((N_PAD,), jnp.int32),
        ],
    )(ht)


def _sc_agg_body(pk_hbm, al_hbm, hpk_hbm, zt_hbm,
                 h_v, z_v, pa_v, aa_v, pb_v, ab_v,
                 sem_a, sem_b):

    wid = _wid()

    def issue(cid, pv, av, sem):
        base = cid * CHUNK
        pltpu.async_copy(pk_hbm.at[pl.ds(base, CHUNK)], pv, sem)
        pltpu.async_copy(al_hbm.at[pl.ds(base, CHUNK)], av, sem)

    def drain(cid, pv, av, sem):
        base = cid * CHUNK
        pltpu.make_async_copy(pk_hbm.at[pl.ds(base, CHUNK)], pv, sem).wait()
        pltpu.make_async_copy(al_hbm.at[pl.ds(base, CHUNK)], av, sem).wait()

    def process(pv_ref, av_ref):
        @plsc.parallel_loop(0, CHUNK, step=L, unroll=2)
        def _(i):
            pv = pv_ref[pl.ds(i, L)]
            sv = lax.shift_right_logical(pv, 14)
            dv = pv & 16383
            av = av_ref[pl.ds(i, L)]
            for c2 in range(CPS // 2):
                cvec = jnp.full((L,), c2, jnp.int32)
                g32 = plsc.load_gather(h_v, [cvec, dv])
                ha, hb = plsc.unpack(plsc.bitcast(g32, jnp.bfloat16),
                                     format=plsc.PackFormat.INTERLEAVED)
                plsc.addupdate_scatter(z_v, [sv + (2 * c2) * N_PAD], ha * av)
                plsc.addupdate_scatter(z_v, [sv + (2 * c2 + 1) * N_PAD],
                                       hb * av)

    c0 = wid * CPS
    pltpu.sync_copy(hpk_hbm.at[pl.ds(wid * (CPS // 2), CPS // 2)], h_v)

    @pl.loop(0, N_PAD * CPS, step=L)
    def _(i):
        z_v[pl.ds(i, L)] = jnp.zeros((L,), jnp.float32)

    issue(0, pa_v, aa_v, sem_a)

    @pl.loop(0, NCH, step=2)
    def _(ci):
        issue(ci + 1, pb_v, ab_v, sem_b)
        drain(ci, pa_v, aa_v, sem_a)
        process(pa_v, aa_v)

        @pl.when(ci + 2 < NCH)
        def _():
            issue(ci + 2, pa_v, aa_v, sem_a)

        drain(ci + 1, pb_v, ab_v, sem_b)
        process(pb_v, ab_v)

    for c in range(CPS):
        pltpu.sync_copy(z_v.at[pl.ds(c * N_PAD, N_PAD)], zt_hbm.at[c0 + c])


def _sc_agg(pk, alpha, hpk):
    return pl.kernel(
        _sc_agg_body,
        out_type=jax.ShapeDtypeStruct((D, N_PAD), jnp.float32),
        mesh=_mesh,
        compiler_params=_sc_params_untiled,
        scratch_types=[
            pltpu.VMEM((CPS // 2, N_PAD), jnp.int32),
            pltpu.VMEM((N_PAD * CPS,), jnp.float32),
            pltpu.VMEM((CHUNK,), jnp.int32),
            pltpu.VMEM((CHUNK,), jnp.float32),
            pltpu.VMEM((CHUNK,), jnp.int32),
            pltpu.VMEM((CHUNK,), jnp.float32),
            pltpu.SemaphoreType.DMA,
            pltpu.SemaphoreType.DMA,
        ],
    )(pk, alpha, hpk)


def _tc_tail_body(zt_ref, x_ref, h_ref, cnt_ref, dp_ref, g_ref, b_ref, o_ref):
    den = jnp.sum(dp_ref[...].T, axis=1, keepdims=True)
    z = zt_ref[...].T / (den + 1e-16)
    cnt = jnp.sum(cnt_ref[...].T, axis=1, keepdims=True)
    zb = jnp.where(cnt > 0.0, z, h_ref[...])
    zr = zb + x_ref[...]
    mu = jnp.mean(zr, axis=1, keepdims=True)
    d = zr - mu
    var = jnp.mean(d * d, axis=1, keepdims=True)
    zn = d * lax.rsqrt(var + 1e-5) * g_ref[...] + b_ref[...]
    o_ref[...] = jnp.where(zn > 0.0, zn, jnp.exp(zn) - 1.0)


def _tc_tail(zt, x, h, cpart, dpart, gamma, beta):
    blk = 1024
    return pl.pallas_call(
        _tc_tail_body,
        grid=(N_PAD // blk,),
        in_specs=[
            pl.BlockSpec((D, blk), lambda i: (0, i)),
            pl.BlockSpec((blk, D), lambda i: (i, 0)),
            pl.BlockSpec((blk, D), lambda i: (i, 0)),
            pl.BlockSpec((NW, blk), lambda i: (0, i)),
            pl.BlockSpec((NW, blk), lambda i: (0, i)),
            pl.BlockSpec((1, D), lambda i: (0, 0)),
            pl.BlockSpec((1, D), lambda i: (0, 0)),
        ],
        out_specs=pl.BlockSpec((blk, D), lambda i: (i, 0)),
        out_shape=jax.ShapeDtypeStruct((N, D), jnp.float32),
    )(zt, x, h, cpart, dpart, gamma, beta)


@jax.jit
def kernel(X, M_ei, M_w, W, a_src, a_dst, gamma, beta):
    src = M_ei[0]
    dst = M_ei[1]
    pad = E_PAD - E
    src_p = jnp.concatenate([src, jnp.full((pad,), N, jnp.int32)])
    dst_p = jnp.concatenate([dst, jnp.zeros((pad,), jnp.int32)])
    mw_p = jnp.concatenate([M_w, jnp.zeros((pad,), jnp.float32)])

    es, ed = _tc_proj(X, W, a_src[:, None], a_dst[:, None])
    H, HT = _tc_head(X, W)
    es_p = jnp.pad(es[:, 0], (0, N_PAD - N))
    ed_p = jnp.pad(ed[:, 0], (0, N_PAD - N))

    ex, pk, dpart, cpart = _sc_edge(src_p, dst_p, mw_p, es_p, ed_p)
    HPK = _sc_pack(HT)
    zt = _sc_agg(pk, ex, HPK)
    return _tc_tail(zt, X, H, cpart, dpart, gamma[None, :], beta[None, :])


# speedup vs baseline: 1.1314x; 1.0001x over previous
import dataclasses
import functools
import jax
import jax.numpy as jnp
from jax import lax
from jax.experimental import pallas as pl
from jax.experimental.pallas import tpu as pltpu
from jax.experimental.pallas import tpu_sc as plsc

N = 10000
E = 160000
D = 256
LRELU = 0.2

NC, NS, L = 2, 16, 16
NW = NC * NS
CHUNK = 1408
NCH = 114
E_PAD = NCH * CHUNK
N_PAD = 10240
CPS = 8

_mesh = plsc.VectorSubcoreMesh(core_axis_name="c", subcore_axis_name="s")

_sc_params = pltpu.CompilerParams()
if "needs_layout_passes" in pltpu.CompilerParams.__dataclass_fields__:
    _sc_params = dataclasses.replace(_sc_params, needs_layout_passes=False)
_sc_params_untiled = dataclasses.replace(_sc_params, use_tc_tiling_on_sc=False)


def _wid():
    return lax.axis_index("s") * NC + lax.axis_index("c")


def _tc_proj_body(x_ref, w_ref, asrc_ref, adst_ref, es_ref, ed_ref):
    wa = jnp.dot(w_ref[...], asrc_ref[...],
                 preferred_element_type=jnp.float32,
                 precision=lax.Precision.HIGHEST)
    wb = jnp.dot(w_ref[...], adst_ref[...],
                 preferred_element_type=jnp.float32,
                 precision=lax.Precision.HIGHEST)
    x = x_ref[...]
    es_ref[...] = jnp.dot(x, wa, preferred_element_type=jnp.float32,
                          precision=lax.Precision.HIGHEST)
    ed_ref[...] = jnp.dot(x, wb, preferred_element_type=jnp.float32,
                          precision=lax.Precision.HIGHEST)


def _tc_proj(x, w, asrc, adst):
    blk = 1024
    return pl.pallas_call(
        _tc_proj_body,
        grid=(N_PAD // blk,),
        in_specs=[
            pl.BlockSpec((blk, D), lambda i: (i, 0)),
            pl.BlockSpec((D, D), lambda i: (0, 0)),
            pl.BlockSpec((D, 1), lambda i: (0, 0)),
            pl.BlockSpec((D, 1), lambda i: (0, 0)),
        ],
        out_specs=[
            pl.BlockSpec((blk, 1), lambda i: (i, 0)),
            pl.BlockSpec((blk, 1), lambda i: (i, 0)),
        ],
        out_shape=[
            jax.ShapeDtypeStruct((N, 1), jnp.float32),
            jax.ShapeDtypeStruct((N, 1), jnp.float32),
        ],
    )(x, w, asrc, adst)


def _tc_head_body(x_ref, w_ref, h_ref, ht_ref):
    h = jnp.dot(x_ref[...], w_ref[...],
                preferred_element_type=jnp.float32,
                precision=lax.Precision.HIGHEST)
    h_ref[...] = h
    ht_ref[...] = h.T


def _tc_head(x, w):
    blk = 1024
    return pl.pallas_call(
        _tc_head_body,
        grid=(N_PAD // blk,),
        in_specs=[
            pl.BlockSpec((blk, D), lambda i: (i, 0)),
            pl.BlockSpec((D, D), lambda i: (0, 0)),
        ],
        out_specs=[
            pl.BlockSpec((blk, D), lambda i: (i, 0)),
            pl.BlockSpec((D, blk), lambda i: (0, i)),
        ],
        out_shape=[
            jax.ShapeDtypeStruct((N, D), jnp.float32),
            jax.ShapeDtypeStruct((D, N_PAD), jnp.float32),
        ],
    )(x, w)


_NJ = -(-NCH // NW)


def _sc_edge_body(src_hbm, dst_hbm, mw_hbm, es_hbm, ed_hbm,
                  ex_hbm, pk_hbm, dpart_hbm, cpart_hbm,
                  es_v, ed_v, den_v, cnt_v, *rest):
    bufs = [rest[5 * j:5 * j + 5] for j in range(_NJ)]
    sem_t = rest[5 * _NJ]
    sems_i = rest[5 * _NJ + 1:5 * _NJ + 1 + _NJ]
    sem_o = rest[5 * _NJ + 1 + _NJ]
    wid = _wid()

    pltpu.async_copy(es_hbm, es_v, sem_t)
    pltpu.async_copy(ed_hbm, ed_v, sem_t)

    for j in range(_NJ):
        cid = wid + j * NW
        src_v, dst_v, mw_v, _, _ = bufs[j]

        @pl.when(cid < NCH)
        def _():
            base = cid * CHUNK
            pltpu.async_copy(src_hbm.at[pl.ds(base, CHUNK)], src_v, sems_i[j])
            pltpu.async_copy(dst_hbm.at[pl.ds(base, CHUNK)], dst_v, sems_i[j])
            pltpu.async_copy(mw_hbm.at[pl.ds(base, CHUNK)], mw_v, sems_i[j])

    @pl.loop(0, N_PAD, step=L)
    def _(i):
        zero = jnp.zeros((L,), jnp.float32)
        den_v[pl.ds(i, L)] = zero
        cnt_v[pl.ds(i, L)] = zero

    pltpu.make_async_copy(es_hbm, es_v, sem_t).wait()
    pltpu.make_async_copy(ed_hbm, ed_v, sem_t).wait()

    for j in range(_NJ):
        cid = wid + j * NW
        src_v, dst_v, mw_v, ex_v, pk_v = bufs[j]

        @pl.when(cid < NCH)
        def _():
            base = cid * CHUNK
            pltpu.make_async_copy(src_hbm.at[pl.ds(base, CHUNK)], src_v,
                                  sems_i[j]).wait()
            pltpu.make_async_copy(dst_hbm.at[pl.ds(base, CHUNK)], dst_v,
                                  sems_i[j]).wait()
            pltpu.make_async_copy(mw_hbm.at[pl.ds(base, CHUNK)], mw_v,
                                  sems_i[j]).wait()

            @plsc.parallel_loop(0, CHUNK, step=L)
            def _(i):
                sv = src_v[pl.ds(i, L)]
                dv = dst_v[pl.ds(i, L)]
                pk_v[pl.ds(i, L)] = sv * 16384 + dv
                e = plsc.load_gather(es_v, [sv]) + plsc.load_gather(ed_v, [dv])
                e = jnp.where(e > 0, e, LRELU * e)
                ex = jnp.exp(mw_v[pl.ds(i, L)] * e)
                ex_v[pl.ds(i, L)] = ex
                plsc.addupdate_scatter(den_v, [sv], ex)
                plsc.addupdate_scatter(cnt_v, [sv], jnp.ones((L,), jnp.float32))

            pltpu.async_copy(ex_v, ex_hbm.at[pl.ds(base, CHUNK)], sem_o)
            pltpu.async_copy(pk_v, pk_hbm.at[pl.ds(base, CHUNK)], sem_o)

    pltpu.sync_copy(den_v, dpart_hbm.at[wid])
    pltpu.sync_copy(cnt_v, cpart_hbm.at[wid])

    for j in range(_NJ):
        cid = wid + j * NW
        _, _, _, ex_v, pk_v = bufs[j]

        @pl.when(cid < NCH)
        def _():
            base = cid * CHUNK
            pltpu.make_async_copy(ex_v, ex_hbm.at[pl.ds(base, CHUNK)],
                                  sem_o).wait()
            pltpu.make_async_copy(pk_v, pk_hbm.at[pl.ds(base, CHUNK)],
                                  sem_o).wait()


def _sc_edge(src, dst, mw, es, ed):
    buf_types = []
    for _ in range(_NJ):
        buf_types += [
            pltpu.VMEM((CHUNK,), jnp.int32),
            pltpu.VMEM((CHUNK,), jnp.int32),
            pltpu.VMEM((CHUNK,), jnp.float32),
            pltpu.VMEM((CHUNK,), jnp.float32),
            pltpu.VMEM((CHUNK,), jnp.int32),
        ]
    return pl.kernel(
        _sc_edge_body,
        out_type=[
            jax.ShapeDtypeStruct((E_PAD,), jnp.float32),
            jax.ShapeDtypeStruct((E_PAD,), jnp.int32),
            jax.ShapeDtypeStruct((NW, N_PAD), jnp.float32),
            jax.ShapeDtypeStruct((NW, N_PAD), jnp.float32),
        ],
        mesh=_mesh,
        compiler_params=_sc_params,
        scratch_types=[
            pltpu.VMEM((N_PAD,), jnp.float32),
            pltpu.VMEM((N_PAD,), jnp.float32),
            pltpu.VMEM((N_PAD,), jnp.float32),
            pltpu.VMEM((N_PAD,), jnp.float32),
        ] + buf_types + [pltpu.SemaphoreType.DMA] * (_NJ + 2),
    )(src, dst, mw, es, ed)


def _sc_pack_body(ht_hbm, hpk_hbm, a_v, b_v, q_v):
    wid = _wid()
    for r in range(CPS // 2):
        k = wid * (CPS // 2) + r
        pltpu.sync_copy(ht_hbm.at[2 * k], a_v)
        pltpu.sync_copy(ht_hbm.at[2 * k + 1], b_v)

        @plsc.parallel_loop(0, N_PAD, step=L)
        def _(i):
            pkd = plsc.pack(a_v[pl.ds(i, L)], b_v[pl.ds(i, L)],
                            format=plsc.PackFormat.INTERLEAVED)
            q_v[pl.ds(i, L)] = plsc.bitcast(pkd, jnp.int32)

        pltpu.sync_copy(q_v, hpk_hbm.at[k])


def _sc_pack(ht):
    return pl.kernel(
        _sc_pack_body,
        out_type=jax.ShapeDtypeStruct((D // 2, N_PAD), jnp.int32),
        mesh=_mesh,
        compiler_params=_sc_params,
        scratch_types=[
            pltpu.VMEM((N_PAD,), jnp.float32),
            pltpu.VMEM((N_PAD,), jnp.float32),
            pltpu.VMEM((N_PAD,), jnp.int32),
        ],
    )(ht)


def _sc_agg_body(pk_hbm, al_hbm, hpk_hbm, zt_hbm,
                 h_v, z_v, pa_v, aa_v, pb_v, ab_v,
                 sem_a, sem_b):

    wid = _wid()

    def issue(cid, pv, av, sem):
        base = cid * CHUNK
        pltpu.async_copy(pk_hbm.at[pl.ds(base, CHUNK)], pv, sem)
        pltpu.async_copy(al_hbm.at[pl.ds(base, CHUNK)], av, sem)

    def drain(cid, pv, av, sem):
        base = cid * CHUNK
        pltpu.make_async_copy(pk_hbm.at[pl.ds(base, CHUNK)], pv, sem).wait()
        pltpu.make_async_copy(al_hbm.at[pl.ds(base, CHUNK)], av, sem).wait()

    def process(pv_ref, av_ref):
        @plsc.parallel_loop(0, CHUNK, step=L, unroll=2)
        def _(i):
            pv = pv_ref[pl.ds(i, L)]
            sv = lax.shift_right_logical(pv, 14)
            dv = pv & 16383
            av = av_ref[pl.ds(i, L)]
            for c2 in range(CPS // 2):
                cvec = jnp.full((L,), c2, jnp.int32)
                g32 = plsc.load_gather(h_v, [cvec, dv])
                ha, hb = plsc.unpack(plsc.bitcast(g32, jnp.bfloat16),
                                     format=plsc.PackFormat.INTERLEAVED)
                plsc.addupdate_scatter(z_v, [sv + (2 * c2) * N_PAD], ha * av)
                plsc.addupdate_scatter(z_v, [sv + (2 * c2 + 1) * N_PAD],
                                       hb * av)

    c0 = wid * CPS
    pltpu.sync_copy(hpk_hbm.at[pl.ds(wid * (CPS // 2), CPS // 2)], h_v)

    @pl.loop(0, N_PAD * CPS, step=L)
    def _(i):
        z_v[pl.ds(i, L)] = jnp.zeros((L,), jnp.float32)

    issue(0, pa_v, aa_v, sem_a)

    @pl.loop(0, NCH, step=2)
    def _(ci):
        issue(ci + 1, pb_v, ab_v, sem_b)
        drain(ci, pa_v, aa_v, sem_a)
        process(pa_v, aa_v)

        @pl.when(ci + 2 < NCH)
        def _():
            issue(ci + 2, pa_v, aa_v, sem_a)

        drain(ci + 1, pb_v, ab_v, sem_b)
        process(pb_v, ab_v)

    for c in range(CPS):
        pltpu.sync_copy(z_v.at[pl.ds(c * N_PAD, N_PAD)], zt_hbm.at[c0 + c])


def _sc_agg(pk, alpha, hpk):
    return pl.kernel(
        _sc_agg_body,
        out_type=jax.ShapeDtypeStruct((D, N_PAD), jnp.float32),
        mesh=_mesh,
        compiler_params=_sc_params_untiled,
        scratch_types=[
            pltpu.VMEM((CPS // 2, N_PAD), jnp.int32),
            pltpu.VMEM((N_PAD * CPS,), jnp.float32),
            pltpu.VMEM((CHUNK,), jnp.int32),
            pltpu.VMEM((CHUNK,), jnp.float32),
            pltpu.VMEM((CHUNK,), jnp.int32),
            pltpu.VMEM((CHUNK,), jnp.float32),
            pltpu.SemaphoreType.DMA,
            pltpu.SemaphoreType.DMA,
        ],
    )(pk, alpha, hpk)


def _tc_tail_body(zt_ref, x_ref, h_ref, cnt_ref, dp_ref, g_ref, b_ref, o_ref):
    den = jnp.sum(dp_ref[...].T, axis=1, keepdims=True)
    z = zt_ref[...].T / (den + 1e-16)
    cnt = jnp.sum(cnt_ref[...].T, axis=1, keepdims=True)
    zb = jnp.where(cnt > 0.0, z, h_ref[...])
    zr = zb + x_ref[...]
    mu = jnp.mean(zr, axis=1, keepdims=True)
    d = zr - mu
    var = jnp.mean(d * d, axis=1, keepdims=True)
    zn = d * lax.rsqrt(var + 1e-5) * g_ref[...] + b_ref[...]
    o_ref[...] = jnp.where(zn > 0.0, zn, jnp.exp(zn) - 1.0)


def _tc_tail(zt, x, h, cpart, dpart, gamma, beta):
    blk = 1024
    return pl.pallas_call(
        _tc_tail_body,
        grid=(N_PAD // blk,),
        in_specs=[
            pl.BlockSpec((D, blk), lambda i: (0, i)),
            pl.BlockSpec((blk, D), lambda i: (i, 0)),
            pl.BlockSpec((blk, D), lambda i: (i, 0)),
            pl.BlockSpec((NW, blk), lambda i: (0, i)),
            pl.BlockSpec((NW, blk), lambda i: (0, i)),
            pl.BlockSpec((1, D), lambda i: (0, 0)),
            pl.BlockSpec((1, D), lambda i: (0, 0)),
        ],
        out_specs=pl.BlockSpec((blk, D), lambda i: (i, 0)),
        out_shape=jax.ShapeDtypeStruct((N, D), jnp.float32),
    )(zt, x, h, cpart, dpart, gamma, beta)


@jax.jit
def kernel(X, M_ei, M_w, W, a_src, a_dst, gamma, beta):
    src = M_ei[0]
    dst = M_ei[1]
    pad = E_PAD - E
    src_p = jnp.concatenate([src, jnp.full((pad,), N, jnp.int32)])
    dst_p = jnp.concatenate([dst, jnp.zeros((pad,), jnp.int32)])
    mw_p = jnp.concatenate([M_w, jnp.zeros((pad,), jnp.float32)])

    es, ed = _tc_proj(X, W, a_src[:, None], a_dst[:, None])
    H, HT = _tc_head(X, W)
    es_p = jnp.pad(es[:, 0], (0, N_PAD - N))
    ed_p = jnp.pad(ed[:, 0], (0, N_PAD - N))

    ex, pk, dpart, cpart = _sc_edge(src_p, dst_p, mw_p, es_p, ed_p)
    HPK = _sc_pack(HT)
    zt = _sc_agg(pk, ex, HPK)
    return _tc_tail(zt, X, H, cpart, dpart, gamma[None, :], beta[None, :])
